# Initial kernel scaffold; baseline (speedup 1.0000x reference)
import jax
import jax.numpy as jnp
from jax.experimental import pallas as pl


def kernel(input_seq, table):
    raise NotImplementedError("write your pallas kernel here")


# speedup vs baseline: 1.0530x; 1.0530x over previous
import functools

import jax
import jax.numpy as jnp
import numpy as np
from jax import lax
from jax.experimental import pallas as pl
from jax.experimental.pallas import tpu as pltpu
from jax.experimental.pallas import tpu_sc as plsc

_NC = 2
_NS = 16
_NW = _NC * _NS
_L = 16


def _pos_encoding_np(d_model, seq_len):
    x = np.linspace(0, d_model - 1, d_model)
    y = np.linspace(0, seq_len - 1, seq_len)
    X, Y = np.meshgrid(x, y)
    z_even = np.sin(Y / np.power(10000, X / d_model))
    z_odd = np.cos(Y / np.power(10000, X / d_model))
    Z = z_odd.copy()
    Z[:, 0::2] = z_even[:, 0::2]
    return Z.astype(np.float32)


@functools.cache
def _make_sc_kernel(batch, seq_len, vocab, d_model):
    s_blk = seq_len // _NW
    chunk = min(s_blk, 64)
    n_chunks = s_blk // chunk

    mesh = plsc.VectorSubcoreMesh(
        core_axis_name="c", subcore_axis_name="s",
        num_cores=_NC, num_subcores=_NS)

    @functools.partial(
        pl.kernel,
        out_type=jax.ShapeDtypeStruct((batch, seq_len, d_model), jnp.float32),
        mesh=mesh,
        scratch_types=[
            pltpu.VMEM((s_blk, d_model), jnp.float32),
            pltpu.VMEM((s_blk,), jnp.int32),
            pltpu.VMEM((chunk, d_model), jnp.float32),
            pltpu.SemaphoreType.DMA,
        ],
    )
    def sc_kernel(table_hbm, idx_hbm, pos_hbm, out_hbm,
                  pos_v, idx_v, rows_v, sem):
        wid = lax.axis_index("s") * _NC + lax.axis_index("c")
        s0 = wid * s_blk
        pltpu.sync_copy(pos_hbm.at[pl.ds(s0, s_blk)], pos_v)
        for b in range(batch):
            pltpu.sync_copy(idx_hbm.at[b, pl.ds(s0, s_blk)], idx_v)
            for k in range(n_chunks):
                pltpu.async_copy(
                    table_hbm.at[idx_v.at[pl.ds(k * chunk, chunk)]],
                    rows_v, sem).wait()

                def row_body(r, carry, k=k):
                    for j in range(d_model // _L):
                        sl = pl.ds(j * _L, _L)
                        rows_v[r, sl] = rows_v[r, sl] + pos_v[k * chunk + r, sl]
                    return carry

                lax.fori_loop(0, chunk, row_body, 0)
                pltpu.sync_copy(rows_v,
                                out_hbm.at[b, pl.ds(s0 + k * chunk, chunk)])

    return sc_kernel


def kernel(input_seq, table):
    batch, seq_len = input_seq.shape
    vocab, d_model = table.shape
    pos = jnp.asarray(_pos_encoding_np(d_model, seq_len))
    idx = input_seq.astype(jnp.int32)
    sc = _make_sc_kernel(batch, seq_len, vocab, d_model)
    return sc(table, idx, pos)


# speedup vs baseline: 1.1120x; 1.0561x over previous
import functools

import jax
import jax.numpy as jnp
import numpy as np
from jax import lax
from jax.experimental import pallas as pl
from jax.experimental.pallas import tpu as pltpu
from jax.experimental.pallas import tpu_sc as plsc

_NC = 2
_NS = 16
_NW = _NC * _NS
_L = 16


def _pos_encoding_np(d_model, seq_len):
    x = np.linspace(0, d_model - 1, d_model)
    y = np.linspace(0, seq_len - 1, seq_len)
    X, Y = np.meshgrid(x, y)
    z_even = np.sin(Y / np.power(10000, X / d_model))
    z_odd = np.cos(Y / np.power(10000, X / d_model))
    Z = z_odd.copy()
    Z[:, 0::2] = z_even[:, 0::2]
    return Z.astype(np.float32)


@functools.cache
def _make_sc_kernel(batch, seq_len, vocab, d_model):
    s_blk = seq_len // _NW
    chunk = 32
    n_chunks = s_blk // chunk
    nbuf = 3
    steps = [(b, k) for b in range(batch) for k in range(n_chunks)]

    mesh = plsc.VectorSubcoreMesh(
        core_axis_name="c", subcore_axis_name="s",
        num_cores=_NC, num_subcores=_NS)

    @functools.partial(
        pl.kernel,
        out_type=jax.ShapeDtypeStruct((batch, seq_len, d_model), jnp.float32),
        mesh=mesh,
        scratch_types=[
            pltpu.VMEM((s_blk, d_model), jnp.float32),
            pltpu.VMEM((batch, s_blk), jnp.int32),
            pltpu.VMEM((nbuf, chunk, d_model), jnp.float32),
            pltpu.SemaphoreType.DMA,
            pltpu.SemaphoreType.DMA,
        ],
    )
    def sc_kernel(table_hbm, idx_hbm, pos_hbm, out_hbm,
                  pos_v, idx_v, rows_v, gsem, ssem):
        wid = lax.axis_index("s") * _NC + lax.axis_index("c")
        s0 = wid * s_blk
        pltpu.sync_copy(pos_hbm.at[pl.ds(s0, s_blk)], pos_v)
        for b in range(batch):
            pltpu.sync_copy(idx_hbm.at[b, pl.ds(s0, s_blk)], idx_v.at[b])

        def start_gather(s):
            b, k = steps[s]
            return pltpu.async_copy(
                table_hbm.at[idx_v.at[b, pl.ds(k * chunk, chunk)]],
                rows_v.at[s % nbuf], gsem)

        g_pending = [start_gather(0), start_gather(1)]
        s_pending = []
        for s, (b, k) in enumerate(steps):
            g_pending.pop(0).wait()
            buf = s % nbuf

            def row_body(r, carry, k=k, buf=buf):
                for j in range(d_model // _L):
                    sl = pl.ds(j * _L, _L)
                    plsc.addupdate(rows_v.at[buf, r, sl],
                                   pos_v[k * chunk + r, sl])
                return carry

            lax.fori_loop(0, chunk, row_body, 0)
            s_pending.append(pltpu.async_copy(
                rows_v.at[buf],
                out_hbm.at[b, pl.ds(s0 + k * chunk, chunk)], ssem))
            if s + 2 < len(steps):
                while len(s_pending) > 1:
                    s_pending.pop(0).wait()
                g_pending.append(start_gather(s + 2))
        while s_pending:
            s_pending.pop(0).wait()

    return sc_kernel


def kernel(input_seq, table):
    batch, seq_len = input_seq.shape
    vocab, d_model = table.shape
    pos = jnp.asarray(_pos_encoding_np(d_model, seq_len))
    idx = input_seq.astype(jnp.int32)
    sc = _make_sc_kernel(batch, seq_len, vocab, d_model)
    return sc(table, idx, pos)
